# baseline (speedup 1.0000x reference)
import functools

import jax
import jax.numpy as jnp
import numpy as np
from jax.experimental import pallas as pl
from jax.experimental.pallas import tpu as pltpu

_C = 64
_CC = 128
_BP = 4
_EPS = 1e-5
_GROUPS = 32

_XP = 255
_P1 = 225
_M2 = 195
_M3 = 48


def _np_masks():
    m1 = np.zeros((4, _P1), np.float32)
    for a in range(2):
        for b in range(2):
            for u in range(15):
                for v in range(15):
                    r, c = 2 * u + a - 1, 2 * v + b - 1
                    if 0 <= r <= 25 and 0 <= c <= 25:
                        m1[a * 2 + b, u * 15 + v] = 1.0
    m2 = np.zeros((_M2,), np.float32)
    for i in range(13):
        for j in range(13):
            m2[i * 15 + j] = 1.0
    m3 = np.zeros((_M3,), np.float32)
    for i in range(6):
        for j in range(6):
            m3[i * 8 + j] = 1.0
    return m1, m2, m3


def _sel(mask_vals, bp, rows):
    s = np.zeros((bp, bp * rows), np.float32)
    for b in range(bp):
        s[b, b * rows:(b + 1) * rows] = mask_vals
    return s


def _blockdiag(a):
    c = a.shape[0]
    out = np.zeros((2 * c, 2 * a.shape[1]), np.float32)
    out[:c, :a.shape[1]] = a
    out[c:, a.shape[1]:] = a
    return out


def _fused_kernel(x_ref, w1_ref, w2_ref, w3_ref, b1_ref, b2_ref, b3_ref,
                  g1_ref, bb1_ref, g2_ref, bb2_ref, g3_ref, bb3_ref,
                  fcw_ref, fcb_ref, m1_ref, m2_ref, m3_ref,
                  a1_ref, a2_ref, a3_ref, s1_ref, s2_ref, s3_ref, sp_ref,
                  o_ref, y1_ref, z_ref, *, bp, eps):
    f32 = jnp.float32
    w1 = w1_ref[...]
    b1 = b1_ref[...]

    for a in range(2):
        for bph in range(2):
            cols = []
            for half in range(2):
                for di in range(3):
                    for dj in range(3):
                        pi, qi = (a + di) & 1, (a + di) >> 1
                        pj, qj = (bph + dj) & 1, (bph + dj) >> 1
                        st = qi * 15 + qj
                        lane = half * 4 + pi * 2 + pj
                        cols.append(x_ref[:, st:st + _P1, lane:lane + 1])
            xcol = jnp.concatenate(cols, axis=-1).reshape(bp * _P1, 18)
            acc = jnp.dot(xcol, w1, preferred_element_type=f32)
            y1_ref[:, a * 2 + bph] = acc.reshape(bp, _P1, _CC) + b1

    s = None
    for p in range(4):
        t = jnp.dot(s1_ref[p], y1_ref[:, p].reshape(bp * _P1, _CC),
                    preferred_element_type=f32)
        s = t if s is None else s + t
    mean1 = jnp.dot(s, a1_ref[...], preferred_element_type=f32)
    v = None
    for p in range(4):
        d = y1_ref[:, p] - mean1[:, None, :]
        t = jnp.dot(s1_ref[p], (d * d).reshape(bp * _P1, _CC),
                    preferred_element_type=f32)
        v = t if v is None else v + t
    var1 = jnp.dot(v, a1_ref[...], preferred_element_type=f32)
    sc1 = jax.lax.rsqrt(var1 + eps)
    g1 = g1_ref[...]
    bb1 = bb1_ref[...]
    for p in range(4):
        y = (y1_ref[:, p] - mean1[:, None, :]) * sc1[:, None, :] * g1 + bb1
        y1_ref[:, p] = jnp.maximum(y, 0.0) * m1_ref[p]

    acc2 = None
    for ki in range(4):
        cat = []
        for kj in range(4):
            p = (ki & 1) * 2 + (kj & 1)
            st = (ki >> 1) * 15 + (kj >> 1)
            cat.append(y1_ref[:, p, st:st + _M2, :].reshape(bp * _M2, _CC))
        xt = jnp.concatenate(cat, axis=-1)
        d = jnp.dot(xt, w2_ref[ki], preferred_element_type=f32)
        acc2 = d if acc2 is None else acc2 + d
    raw2 = acc2.reshape(bp, _M2, _CC) + b2_ref[...]
    s2v = jnp.dot(s2_ref[...], raw2.reshape(bp * _M2, _CC),
                  preferred_element_type=f32)
    mean2 = jnp.dot(s2v, a2_ref[...], preferred_element_type=f32)
    d2 = raw2 - mean2[:, None, :]
    v2 = jnp.dot(s2_ref[...], (d2 * d2).reshape(bp * _M2, _CC),
                 preferred_element_type=f32)
    var2 = jnp.dot(v2, a2_ref[...], preferred_element_type=f32)
    sc2 = jax.lax.rsqrt(var2 + eps)
    y2 = d2 * sc2[:, None, :] * g2_ref[...] + bb2_ref[...]
    y2 = jnp.maximum(y2, 0.0) * m2_ref[...]

    z_ref[...] = jnp.zeros((bp, 16, 16, _CC), f32)
    z_ref[:, 1:14, 1:16, :] = y2.reshape(bp, 13, 15, _CC)
    z6 = z_ref[...].reshape(bp, 8, 2, 8, 2, _CC)
    ph3 = []
    for c in range(2):
        for d in range(2):
            ph3.append(z6[:, :, c, :, d, :].reshape(bp, 64, _CC))

    acc3 = None
    for ki in range(4):
        cat = []
        for kj in range(4):
            ph = ph3[(ki & 1) * 2 + (kj & 1)]
            st = (ki >> 1) * 8 + (kj >> 1)
            cat.append(ph[:, st:st + _M3, :].reshape(bp * _M3, _CC))
        xt = jnp.concatenate(cat, axis=-1)
        d = jnp.dot(xt, w3_ref[ki], preferred_element_type=f32)
        acc3 = d if acc3 is None else acc3 + d
    raw3 = acc3.reshape(bp, _M3, _CC) + b3_ref[...]
    s3v = jnp.dot(s3_ref[...], raw3.reshape(bp * _M3, _CC),
                  preferred_element_type=f32)
    mean3 = jnp.dot(s3v, a3_ref[...], preferred_element_type=f32)
    d3 = raw3 - mean3[:, None, :]
    v3 = jnp.dot(s3_ref[...], (d3 * d3).reshape(bp * _M3, _CC),
                 preferred_element_type=f32)
    var3 = jnp.dot(v3, a3_ref[...], preferred_element_type=f32)
    sc3 = jax.lax.rsqrt(var3 + eps)
    y3 = d3 * sc3[:, None, :] * g3_ref[...] + bb3_ref[...]
    y3 = jnp.maximum(y3, 0.0) * m3_ref[...]
    pooled = jnp.dot(sp_ref[...], y3.reshape(bp * _M3, _CC),
                     preferred_element_type=f32)
    o_ref[...] = jnp.dot(pooled, fcw_ref[...],
                         preferred_element_type=f32) + fcb_ref[...]


def _bcast(shape):
    zeros = (0,) * len(shape)
    return pl.BlockSpec(shape, lambda i, _z=zeros: _z)


def kernel(x, w1, b1, w2, b2, w3, b3, gn1_g, gn1_b, gn2_g, gn2_b,
           gn3_g, gn3_b, fc_w, fc_b):
    n = x.shape[0]
    c = _C
    cc = _CC
    bp = _BP
    np_ = np
    xp = jnp.pad(x[:, 0], ((0, 0), (1, 5), (1, 1)))
    xph = xp.reshape(n, 17, 2, 15, 2).transpose(0, 2, 4, 1, 3)
    xph = xph.reshape(n, 4, _XP).transpose(0, 2, 1)
    xph = xph.reshape(n // 2, 2, _XP, 4).transpose(0, 2, 1, 3)
    xph = xph.reshape(n // 2, _XP, 8)

    w1c = np.zeros((18, cc), np.float32)
    w1h = np.asarray(w1).reshape(c, 9).T
    w1c[0:9, 0:c] = w1h
    w1c[9:18, c:cc] = w1h
    w2g = np.transpose(np.asarray(w2), (2, 3, 1, 0))

    def packtaps(wg):
        out = np.zeros((4, 4 * cc, cc), np.float32)
        for ki in range(4):
            for kj in range(4):
                blk = _blockdiag(wg[ki, kj])
                out[ki, kj * cc:(kj + 1) * cc] = blk
        return out

    w2t = packtaps(w2g)
    w3t = packtaps(np.transpose(np.asarray(w3), (2, 3, 1, 0)))

    n_cls = fc_w.shape[0]
    fcw = np.zeros((cc, 2 * n_cls), np.float32)
    fcw_h = np.asarray(fc_w).T
    fcw[0:c, 0:n_cls] = fcw_h
    fcw[c:cc, n_cls:2 * n_cls] = fcw_h
    fcb_h = np.asarray(fc_b)
    fcb = np.concatenate([fcb_h, fcb_h]).reshape(1, 2 * n_cls)

    m1np, m2np, m3np = _np_masks()
    s1 = np.stack([_sel(m1np[p], bp, _P1) for p in range(4)])
    s2 = _sel(m2np, bp, _M2)
    s3 = _sel(m3np, bp, _M3)
    spool = _sel(np.full((_M3,), 1.0 / 36.0, np.float32), bp, _M3)

    cg = c // _GROUPS
    g = np.arange(c) // cg

    def amat(count):
        a = ((g[:, None] == g[None, :]).astype(np.float32)
             / float(cg * count))
        return _blockdiag(a)

    def dup(v):
        vv = np.asarray(v)
        return np.concatenate([vv, vv]).reshape(1, cc)

    m1b = m1np.reshape(4, _P1, 1)
    m2b = m2np.reshape(_M2, 1)
    m3b = m3np.reshape(_M3, 1)

    out = pl.pallas_call(
        functools.partial(_fused_kernel, bp=bp, eps=_EPS),
        grid=(n // 2 // bp,),
        in_specs=[
            pl.BlockSpec((bp, _XP, 8), lambda i: (i, 0, 0)),
            _bcast((18, cc)), _bcast((4, 4 * cc, cc)), _bcast((4, 4 * cc, cc)),
            _bcast((1, cc)), _bcast((1, cc)), _bcast((1, cc)),
            _bcast((1, cc)), _bcast((1, cc)), _bcast((1, cc)),
            _bcast((1, cc)), _bcast((1, cc)), _bcast((1, cc)),
            _bcast((cc, 2 * n_cls)), _bcast((1, 2 * n_cls)),
            _bcast((4, _P1, 1)), _bcast((_M2, 1)), _bcast((_M3, 1)),
            _bcast((cc, cc)), _bcast((cc, cc)), _bcast((cc, cc)),
            _bcast((4, bp, bp * _P1)), _bcast((bp, bp * _M2)),
            _bcast((bp, bp * _M3)), _bcast((bp, bp * _M3)),
        ],
        out_specs=pl.BlockSpec((bp, 2 * n_cls), lambda i: (i, 0)),
        out_shape=jax.ShapeDtypeStruct((n // 2, 2 * n_cls), jnp.float32),
        scratch_shapes=[
            pltpu.VMEM((bp, 4, _P1, cc), jnp.float32),
            pltpu.VMEM((bp, 16, 16, cc), jnp.float32),
        ],
        compiler_params=pltpu.CompilerParams(
            dimension_semantics=("parallel",)),
    )(jnp.asarray(xph), jnp.asarray(w1c), jnp.asarray(w2t), jnp.asarray(w3t),
      jnp.asarray(dup(b1)), jnp.asarray(dup(b2)), jnp.asarray(dup(b3)),
      jnp.asarray(dup(gn1_g)), jnp.asarray(dup(gn1_b)),
      jnp.asarray(dup(gn2_g)), jnp.asarray(dup(gn2_b)),
      jnp.asarray(dup(gn3_g)), jnp.asarray(dup(gn3_b)),
      jnp.asarray(fcw), jnp.asarray(fcb),
      jnp.asarray(m1b), jnp.asarray(m2b), jnp.asarray(m3b),
      jnp.asarray(amat(26 * 26)), jnp.asarray(amat(13 * 13)),
      jnp.asarray(amat(6 * 6)),
      jnp.asarray(s1), jnp.asarray(s2), jnp.asarray(s3), jnp.asarray(spool))
    return out.reshape(n, n_cls)


# speedup vs baseline: 2.0848x; 2.0848x over previous
import functools

import jax
import jax.numpy as jnp
import numpy as np
from jax.experimental import pallas as pl
from jax.experimental.pallas import tpu as pltpu

_C = 64
_CC = 128
_BP = 8
_EPS = 1e-5
_GROUPS = 32

_XP = 255
_P1 = 225
_M2 = 195
_M3 = 48


def _np_masks():
    m1 = np.zeros((4, _P1), np.float32)
    for a in range(2):
        for b in range(2):
            for u in range(15):
                for v in range(15):
                    r, c = 2 * u + a - 1, 2 * v + b - 1
                    if 0 <= r <= 25 and 0 <= c <= 25:
                        m1[a * 2 + b, u * 15 + v] = 1.0
    m2 = np.zeros((_M2,), np.float32)
    for i in range(13):
        for j in range(13):
            m2[i * 15 + j] = 1.0
    m3 = np.zeros((_M3,), np.float32)
    for i in range(6):
        for j in range(6):
            m3[i * 8 + j] = 1.0
    return m1, m2, m3


def _sel(mask_vals, bp, rows):
    s = np.zeros((bp, bp * rows), np.float32)
    for b in range(bp):
        s[b, b * rows:(b + 1) * rows] = mask_vals
    return s


def _blockdiag(a):
    c = a.shape[0]
    out = np.zeros((2 * c, 2 * a.shape[1]), np.float32)
    out[:c, :a.shape[1]] = a
    out[c:, a.shape[1]:] = a
    return out


def _fused_kernel(x_ref, w1_ref, w2_ref, w3_ref, b1_ref, b2_ref, b3_ref,
                  g1_ref, bb1_ref, g2_ref, bb2_ref, g3_ref, bb3_ref,
                  fcw_ref, fcb_ref, m1_ref, m2_ref, m3_ref,
                  a1_ref, a2_ref, a3_ref, s1_ref, s2_ref, s3_ref, sp_ref,
                  o_ref, y1_ref, z_ref, *, bp, eps):
    f32 = jnp.float32
    w1 = w1_ref[...]
    b1 = b1_ref[...]

    s = None
    v = None
    for a in range(2):
        for bph in range(2):
            cols = []
            for half in range(2):
                for di in range(3):
                    for dj in range(3):
                        pi, qi = (a + di) & 1, (a + di) >> 1
                        pj, qj = (bph + dj) & 1, (bph + dj) >> 1
                        st = qi * 15 + qj
                        lane = half * 4 + pi * 2 + pj
                        cols.append(x_ref[:, st:st + _P1, lane:lane + 1])
            xcol = jnp.concatenate(cols, axis=-1).reshape(bp * _P1, 18)
            acc = jnp.dot(xcol, w1, preferred_element_type=f32)
            y = acc + jnp.broadcast_to(b1, (1, _CC))
            p = a * 2 + bph
            t = jnp.dot(s1_ref[p], y, preferred_element_type=f32)
            s = t if s is None else s + t
            t2 = jnp.dot(s1_ref[p], y * y, preferred_element_type=f32)
            v = t2 if v is None else v + t2
            y1_ref[:, p] = y.reshape(bp, _P1, _CC)

    mean1 = jnp.dot(s, a1_ref[...], preferred_element_type=f32)
    ey2 = jnp.dot(v, a1_ref[...], preferred_element_type=f32)
    var1 = ey2 - mean1 * mean1
    sc1 = jax.lax.rsqrt(var1 + eps)
    g1 = g1_ref[...]
    bb1 = bb1_ref[...]
    for p in range(4):
        y = (y1_ref[:, p] - mean1[:, None, :]) * sc1[:, None, :] * g1 + bb1
        y1_ref[:, p] = jnp.maximum(y, 0.0) * m1_ref[p]

    acc2 = None
    for ki in range(4):
        for kj in range(4):
            p = (ki & 1) * 2 + (kj & 1)
            st = (ki >> 1) * 15 + (kj >> 1)
            xt = y1_ref[:, p, st:st + _M2, :].reshape(bp * _M2, _CC)
            d = jnp.dot(xt, w2_ref[ki * 4 + kj], preferred_element_type=f32)
            acc2 = d if acc2 is None else acc2 + d
    raw2 = acc2.reshape(bp, _M2, _CC) + b2_ref[...]
    r2f = raw2.reshape(bp * _M2, _CC)
    s2v = jnp.dot(s2_ref[...], r2f, preferred_element_type=f32)
    v2 = jnp.dot(s2_ref[...], r2f * r2f, preferred_element_type=f32)
    mean2 = jnp.dot(s2v, a2_ref[...], preferred_element_type=f32)
    var2 = jnp.dot(v2, a2_ref[...],
                   preferred_element_type=f32) - mean2 * mean2
    sc2 = jax.lax.rsqrt(var2 + eps)
    y2 = (raw2 - mean2[:, None, :]) * sc2[:, None, :] * g2_ref[...] \
        + bb2_ref[...]
    y2 = jnp.maximum(y2, 0.0) * m2_ref[...]

    z_ref[...] = jnp.zeros((bp, 16, 16, _CC), f32)
    z_ref[:, 1:14, 1:16, :] = y2.reshape(bp, 13, 15, _CC)
    z6 = z_ref[...].reshape(bp, 8, 2, 8, 2, _CC)
    ph3 = []
    for c in range(2):
        for d in range(2):
            ph3.append(z6[:, :, c, :, d, :].reshape(bp, 64, _CC))

    acc3 = None
    for ki in range(4):
        for kj in range(4):
            ph = ph3[(ki & 1) * 2 + (kj & 1)]
            st = (ki >> 1) * 8 + (kj >> 1)
            xt = ph[:, st:st + _M3, :].reshape(bp * _M3, _CC)
            d = jnp.dot(xt, w3_ref[ki * 4 + kj], preferred_element_type=f32)
            acc3 = d if acc3 is None else acc3 + d
    raw3 = acc3.reshape(bp, _M3, _CC) + b3_ref[...]
    r3f = raw3.reshape(bp * _M3, _CC)
    s3v = jnp.dot(s3_ref[...], r3f, preferred_element_type=f32)
    v3 = jnp.dot(s3_ref[...], r3f * r3f, preferred_element_type=f32)
    mean3 = jnp.dot(s3v, a3_ref[...], preferred_element_type=f32)
    var3 = jnp.dot(v3, a3_ref[...],
                   preferred_element_type=f32) - mean3 * mean3
    sc3 = jax.lax.rsqrt(var3 + eps)
    y3 = (raw3 - mean3[:, None, :]) * sc3[:, None, :] * g3_ref[...] \
        + bb3_ref[...]
    y3 = jnp.maximum(y3, 0.0) * m3_ref[...]
    pooled = jnp.dot(sp_ref[...], y3.reshape(bp * _M3, _CC),
                     preferred_element_type=f32)
    o_ref[...] = jnp.dot(pooled, fcw_ref[...],
                         preferred_element_type=f32) + fcb_ref[...]


def _bcast(shape):
    zeros = (0,) * len(shape)
    return pl.BlockSpec(shape, lambda i, _z=zeros: _z)


def kernel(x, w1, b1, w2, b2, w3, b3, gn1_g, gn1_b, gn2_g, gn2_b,
           gn3_g, gn3_b, fc_w, fc_b):
    n = x.shape[0]
    c = _C
    cc = _CC
    bp = _BP
    xp = jnp.pad(x[:, 0], ((0, 0), (1, 5), (1, 1)))
    xph = xp.reshape(n, 17, 2, 15, 2).transpose(0, 2, 4, 1, 3)
    xph = xph.reshape(n, 4, _XP).transpose(0, 2, 1)
    xph = xph.reshape(n // 2, 2, _XP, 4).transpose(0, 2, 1, 3)
    xph = xph.reshape(n // 2, _XP, 8)

    def jbd(a):
        zz = jnp.zeros_like(a)
        return jnp.concatenate(
            [jnp.concatenate([a, zz], axis=1),
             jnp.concatenate([zz, a], axis=1)], axis=0)

    w1h = w1.reshape(c, 9).T
    z9 = jnp.zeros((9, c), jnp.float32)
    w1c = jnp.concatenate(
        [jnp.concatenate([w1h, z9], axis=1),
         jnp.concatenate([z9, w1h], axis=1)], axis=0)

    def packtaps(w):
        wg = jnp.transpose(w, (2, 3, 1, 0))
        return jnp.stack([jbd(wg[ki, kj])
                          for ki in range(4) for kj in range(4)])

    w2t = packtaps(w2)
    w3t = packtaps(w3)

    n_cls = fc_w.shape[0]
    fcw = jbd(fc_w.T)
    fcb = jnp.concatenate([fc_b, fc_b]).reshape(1, 2 * n_cls)

    m1np, m2np, m3np = _np_masks()
    s1 = np.stack([_sel(m1np[p], bp, _P1) for p in range(4)])
    s2 = _sel(m2np, bp, _M2)
    s3 = _sel(m3np, bp, _M3)
    spool = _sel(np.full((_M3,), 1.0 / 36.0, np.float32), bp, _M3)

    cg = c // _GROUPS
    g = np.arange(c) // cg

    def amat(count):
        a = ((g[:, None] == g[None, :]).astype(np.float32)
             / float(cg * count))
        return _blockdiag(a)

    def dup(v):
        return jnp.concatenate([v, v]).reshape(1, cc)

    m1b = m1np.reshape(4, _P1, 1)
    m2b = m2np.reshape(_M2, 1)
    m3b = m3np.reshape(_M3, 1)

    out = pl.pallas_call(
        functools.partial(_fused_kernel, bp=bp, eps=_EPS),
        grid=(n // 2 // bp,),
        in_specs=[
            pl.BlockSpec((bp, _XP, 8), lambda i: (i, 0, 0)),
            _bcast((18, cc)), _bcast((16, cc, cc)), _bcast((16, cc, cc)),
            _bcast((1, cc)), _bcast((1, cc)), _bcast((1, cc)),
            _bcast((1, cc)), _bcast((1, cc)), _bcast((1, cc)),
            _bcast((1, cc)), _bcast((1, cc)), _bcast((1, cc)),
            _bcast((cc, 2 * n_cls)), _bcast((1, 2 * n_cls)),
            _bcast((4, _P1, 1)), _bcast((_M2, 1)), _bcast((_M3, 1)),
            _bcast((cc, cc)), _bcast((cc, cc)), _bcast((cc, cc)),
            _bcast((4, bp, bp * _P1)), _bcast((bp, bp * _M2)),
            _bcast((bp, bp * _M3)), _bcast((bp, bp * _M3)),
        ],
        out_specs=pl.BlockSpec((bp, 2 * n_cls), lambda i: (i, 0)),
        out_shape=jax.ShapeDtypeStruct((n // 2, 2 * n_cls), jnp.float32),
        scratch_shapes=[
            pltpu.VMEM((bp, 4, _P1, cc), jnp.float32),
            pltpu.VMEM((bp, 16, 16, cc), jnp.float32),
        ],
        compiler_params=pltpu.CompilerParams(
            dimension_semantics=("parallel",)),
    )(jnp.asarray(xph), jnp.asarray(w1c), jnp.asarray(w2t), jnp.asarray(w3t),
      jnp.asarray(dup(b1)), jnp.asarray(dup(b2)), jnp.asarray(dup(b3)),
      jnp.asarray(dup(gn1_g)), jnp.asarray(dup(gn1_b)),
      jnp.asarray(dup(gn2_g)), jnp.asarray(dup(gn2_b)),
      jnp.asarray(dup(gn3_g)), jnp.asarray(dup(gn3_b)),
      jnp.asarray(fcw), jnp.asarray(fcb),
      jnp.asarray(m1b), jnp.asarray(m2b), jnp.asarray(m3b),
      jnp.asarray(amat(26 * 26)), jnp.asarray(amat(13 * 13)),
      jnp.asarray(amat(6 * 6)),
      jnp.asarray(s1), jnp.asarray(s2), jnp.asarray(s3), jnp.asarray(spool))
    return out.reshape(n, n_cls)
